# baseline (speedup 1.0000x reference)
import jax
import jax.numpy as jnp
from jax import lax
from jax.experimental import pallas as pl
from jax.experimental.pallas import tpu as pltpu
from jax.experimental.pallas import tpu_sc as plsc

M, K, H, B = 4096, 3, 2048, 64
NC, NS = 2, 16
NW = NC * NS
BPW = B // NW
MK = M * K


def _patch_body(pool, new_r, gidx, sidxg, sidxn, sid, ntok, tok, tok_out,
                gidx_v, sidxg_v, sidxn_v, gbuf, nbuf, tok_v, sid_v, ntok_v,
                sem):
  w = lax.axis_index("s") * NC + lax.axis_index("c")

  pltpu.sync_copy(gidx.at[w], gidx_v)
  pltpu.sync_copy(sidxg.at[w], sidxg_v)
  pltpu.sync_copy(sidxn.at[w], sidxn_v)
  pltpu.async_copy(pool.at[gidx_v], gbuf, sem).wait()
  pltpu.sync_copy(new_r.at[w], nbuf)
  pltpu.async_copy(gbuf, pool.at[sidxg_v], sem).wait()
  pltpu.async_copy(nbuf, pool.at[sidxn_v], sem).wait()

  @pl.when(w == 0)
  def _():
    pltpu.sync_copy(tok, tok_v)
    pltpu.sync_copy(sid, sid_v)
    pltpu.sync_copy(ntok, ntok_v)
    for v in range(B // 16):
      s = sid_v[pl.ds(16 * v, 16)]
      r = s * 3
      g1 = plsc.load_gather(tok_v, [r + 1])
      g2 = plsc.load_gather(tok_v, [r + 2])
      nt = ntok_v[pl.ds(16 * v, 16)]
      plsc.store_scatter(tok_v, [r], g1)
      plsc.store_scatter(tok_v, [r + 1], g2)
      plsc.store_scatter(tok_v, [r + 2], nt)
    pltpu.sync_copy(tok_v, tok_out)


_sc_patch = pl.kernel(
    _patch_body,
    out_type=jax.ShapeDtypeStruct((MK,), jnp.int32),
    mesh=plsc.VectorSubcoreMesh(core_axis_name="c", subcore_axis_name="s"),
    scratch_types=[
        pltpu.VMEM((2 * BPW,), jnp.int32),
        pltpu.VMEM((2 * BPW,), jnp.int32),
        pltpu.VMEM((BPW,), jnp.int32),
        pltpu.VMEM((2 * BPW, H), jnp.float32),
        pltpu.VMEM((BPW, H), jnp.float32),
        pltpu.VMEM((MK,), jnp.int32),
        pltpu.VMEM((B,), jnp.int32),
        pltpu.VMEM((B,), jnp.int32),
        pltpu.SemaphoreType.DMA,
    ],
    compiler_params=pltpu.CompilerParams(needs_layout_passes=False),
    name="mtp_pool_patch_sc",
)


@jax.jit
def kernel(mem_hidden, new_hidden, slot_ids, mem_tokens, new_tokens):
  base = slot_ids.astype(jnp.int32) * 3
  gidx = jnp.stack([base + 1, base + 2], axis=1).reshape(NW, 2 * BPW)
  sidxg = jnp.stack([base, base + 1], axis=1).reshape(NW, 2 * BPW)
  sidxn = (base + 2).reshape(NW, BPW)

  pool = jax.new_ref(mem_hidden.reshape(MK, H))
  tok_out = _sc_patch(pool, new_hidden.reshape(NW, BPW, H), gidx, sidxg,
                      sidxn, slot_ids.astype(jnp.int32), new_tokens,
                      mem_tokens.reshape(MK))
  return pool[...].reshape(M, K, H), tok_out.reshape(M, K)


# speedup vs baseline: 1.0011x; 1.0011x over previous
import jax
import jax.numpy as jnp
from jax import lax
from jax.experimental import pallas as pl
from jax.experimental.pallas import tpu as pltpu
from jax.experimental.pallas import tpu_sc as plsc

M, K, H, B = 4096, 3, 2048, 64
NC, NS = 2, 16
NW = NC * NS
BPW = B // NW
MK = M * K


def _patch_body(pool, new_r, gidx, sidxg, sidxn, sid, ntok, tok, tok_out,
                gidx_v, sidxg_v, sidxn_v, gbuf, nbuf, tok_v, sid_v, ntok_v,
                sem):
  w = lax.axis_index("s") * NC + lax.axis_index("c")

  pltpu.sync_copy(gidx.at[w], gidx_v)
  pltpu.sync_copy(sidxg.at[w], sidxg_v)
  pltpu.sync_copy(sidxn.at[w], sidxn_v)
  pltpu.async_copy(pool.at[gidx_v], gbuf, sem).wait()
  pltpu.sync_copy(new_r.at[w], nbuf)
  pltpu.async_copy(gbuf, pool.at[sidxg_v], sem).wait()
  pltpu.async_copy(nbuf, pool.at[sidxn_v], sem).wait()

  @pl.when(w == 0)
  def _():
    pltpu.sync_copy(tok, tok_v)
    pltpu.sync_copy(sid, sid_v)
    pltpu.sync_copy(ntok, ntok_v)
    for v in range(B // 16):
      s = sid_v[pl.ds(16 * v, 16)]
      r = s * 3
      g1 = plsc.load_gather(tok_v, [r + 1])
      g2 = plsc.load_gather(tok_v, [r + 2])
      nt = ntok_v[pl.ds(16 * v, 16)]
      plsc.store_scatter(tok_v, [r], g1)
      plsc.store_scatter(tok_v, [r + 1], g2)
      plsc.store_scatter(tok_v, [r + 2], nt)
    pltpu.sync_copy(tok_v, tok_out)


_sc_patch = pl.kernel(
    _patch_body,
    out_type=jax.ShapeDtypeStruct((MK,), jnp.int32),
    mesh=plsc.VectorSubcoreMesh(core_axis_name="c", subcore_axis_name="s"),
    scratch_types=[
        pltpu.VMEM((2 * BPW,), jnp.int32),
        pltpu.VMEM((2 * BPW,), jnp.int32),
        pltpu.VMEM((BPW,), jnp.int32),
        pltpu.VMEM((2 * BPW, H), jnp.float32),
        pltpu.VMEM((BPW, H), jnp.float32),
        pltpu.VMEM((MK,), jnp.int32),
        pltpu.VMEM((B,), jnp.int32),
        pltpu.VMEM((B,), jnp.int32),
        pltpu.SemaphoreType.DMA,
    ],
    compiler_params=pltpu.CompilerParams(needs_layout_passes=False),
    name="mtp_pool_patch_sc",
)


@jax.jit
def kernel(mem_hidden, new_hidden, slot_ids, mem_tokens, new_tokens):
  base = slot_ids.astype(jnp.int32) * 3
  gidx = jnp.stack([base + 1, base + 2], axis=1).reshape(NW, 2 * BPW)
  sidxg = jnp.stack([base, base + 1], axis=1).reshape(NW, 2 * BPW)
  sidxn = (base + 2).reshape(NW, BPW)

  pool = jax.new_ref(mem_hidden.reshape(MK, H))
  tok_out = _sc_patch(pool, new_hidden.reshape(NW, BPW, H), gidx, sidxg,
                      sidxn, slot_ids.astype(jnp.int32), new_tokens,
                      mem_tokens.reshape(MK))
  return jax.freeze(pool).reshape(M, K, H), tok_out.reshape(M, K)
